# baseline (speedup 1.0000x reference)
import functools

import jax
import jax.numpy as jnp
from jax import lax
from jax.experimental import pallas as pl
from jax.experimental.pallas import tpu as pltpu
from jax.experimental.pallas import tpu_sc as plsc

NW = 32
LANES = 128
L = 16


def _sc_gather_main_t(idx2d, cell_means, B, D, DM):
    b_per_w = B // NW
    n_chunks = DM // LANES

    mesh = plsc.VectorSubcoreMesh(core_axis_name="c", subcore_axis_name="s")

    b_sub = b_per_w // 2
    IP = LANES + 8
    OP = b_per_w + 8

    @functools.partial(
        pl.kernel,
        mesh=mesh,
        out_type=jax.ShapeDtypeStruct((D, B), jnp.float32),
        compiler_params=pltpu.CompilerParams(needs_layout_passes=False),
        scratch_types=[
            pltpu.VMEM((b_per_w,), jnp.int32),
            *[pltpu.VMEM((b_sub, IP), jnp.float32) for _ in range(2)],
            *[pltpu.VMEM((LANES, OP), jnp.float32) for _ in range(2)],
            *[pltpu.SemaphoreType.DMA for _ in range(4)],
        ],
    )
    def gather_kernel(idx_hbm, table_hbm, outt_hbm, idx_v, *rest):
        ibufs = rest[0:2]
        obufs = rest[2:4]
        gsems = rest[4:6]
        wsems = rest[6:8]
        wid = lax.axis_index("s") * 2 + lax.axis_index("c")
        col0 = wid * b_per_w
        pltpu.sync_copy(idx_hbm.at[wid], idx_v)

        def g_start(c, h):
            pltpu.async_copy(
                table_hbm.at[idx_v.at[pl.ds(h * b_sub, b_sub)],
                             pl.ds(c * LANES, LANES)],
                ibufs[h].at[:, pl.ds(0, LANES)],
                gsems[h],
            )

        def g_wait(c, h):
            pltpu.make_async_copy(
                table_hbm.at[idx_v.at[pl.ds(h * b_sub, b_sub)],
                             pl.ds(c * LANES, LANES)],
                ibufs[h].at[:, pl.ds(0, LANES)],
                gsems[h],
            ).wait()

        def out_at(c):
            return outt_hbm.at[pl.ds(c * LANES, LANES), pl.ds(col0, b_per_w)]

        def w_start(c, q):
            pltpu.async_copy(obufs[q].at[:, pl.ds(0, b_per_w)], out_at(c),
                             wsems[q])

        def w_wait(c, q):
            pltpu.make_async_copy(
                obufs[q].at[:, pl.ds(0, b_per_w)], out_at(c), wsems[q]
            ).wait()

        iota16 = lax.iota(jnp.int32, L)
        perms = [(iota16 + m) & (L - 1) for m in range(L)]

        def transpose(h, q):
            def jbody(j, carry):
                rows = iota16 + (j * L)
                ocols = rows + (h * b_sub)
                for t in range(LANES // L):
                    for m in range(L):
                        cols = perms[m] + (t * L)
                        v = plsc.load_gather(ibufs[h], [rows, cols])
                        plsc.store_scatter(obufs[q], [cols, ocols], v)
                return carry

            lax.fori_loop(0, b_sub // L, jbody, 0)

        g_start(0, 0)
        g_start(0, 1)

        def body(c2, carry):
            for q in range(2):
                c = c2 * 2 + q

                @pl.when(c >= 2)
                def _():
                    w_wait(c - 2, q)

                for h in range(2):
                    g_wait(c, h)
                    transpose(h, q)

                    @pl.when(c + 1 < n_chunks)
                    def _():
                        g_start(c + 1, h)

                w_start(c, q)
            return carry

        lax.fori_loop(0, n_chunks // 2, body, 0)
        w_wait(n_chunks - 2, 0)
        w_wait(n_chunks - 1, 1)

    return gather_kernel(idx2d, cell_means)


def _tc_gather_tail_t(idx, cell_means, outt_main, B, V, D, DM):
    DT = D - DM

    def tail_kernel(idx_ref, tab_ref, _, o_ref):
        onehot = (
            lax.broadcasted_iota(jnp.int32, (V, B), 0) == idx_ref[...]
        ).astype(jnp.float32)
        res = lax.dot_general(
            tab_ref[...],
            onehot,
            (((0,), (0,)), ((), ())),
            preferred_element_type=jnp.float32,
            precision=lax.Precision.HIGHEST,
        )
        o_ref[...] = res[:DT, :]

    return pl.pallas_call(
        tail_kernel,
        grid=(1,),
        in_specs=[
            pl.BlockSpec((1, B), lambda g: (0, 0)),
            pl.BlockSpec((V, LANES), lambda g: (0, DM // LANES)),
            pl.BlockSpec(memory_space=pl.ANY),
        ],
        out_specs=pl.BlockSpec((DT, B), lambda g: (DM // DT, 0)),
        out_shape=jax.ShapeDtypeStruct((D, B), jnp.float32),
        input_output_aliases={2: 0},
    )(idx.reshape(1, B), cell_means, outt_main)


def kernel(cell_index, cell_means):
    B = cell_index.shape[0]
    V, D = cell_means.shape
    DM = (D // LANES) * LANES

    idx = cell_index.astype(jnp.int32)
    idx2d = idx.reshape(NW, B // NW)

    outt = _sc_gather_main_t(idx2d, cell_means, B, D, DM)
    if DM != D:
        outt = _tc_gather_tail_t(idx, cell_means, outt, B, V, D, DM)
    return outt.T


# speedup vs baseline: 2.1201x; 2.1201x over previous
import functools

import jax
import jax.numpy as jnp
from jax import lax
from jax.experimental import pallas as pl
from jax.experimental.pallas import tpu as pltpu
from jax.experimental.pallas import tpu_sc as plsc

NW = 32
NBUF = 4
LANES = 128


def _sc_gather_main(idx2d, cell_means, B, D, DM):
    b_per_w = B // NW

    mesh = plsc.VectorSubcoreMesh(core_axis_name="c", subcore_axis_name="s")

    @functools.partial(
        pl.kernel,
        mesh=mesh,
        out_type=jax.ShapeDtypeStruct((B, D), jnp.float32),
        scratch_types=[
            pltpu.VMEM((NW, b_per_w), jnp.int32),
            *[pltpu.VMEM((1, DM), jnp.float32) for _ in range(NBUF)],
            *[pltpu.SemaphoreType.DMA for _ in range(2 * NBUF)],
        ],
    )
    def gather_kernel(idx_hbm, table_hbm, out_hbm, idx_v, *rest):
        bufs = rest[:NBUF]
        gsems = rest[NBUF : 2 * NBUF]
        wsems = rest[2 * NBUF :]
        wid = lax.axis_index("s") * 2 + lax.axis_index("c")
        base = wid * b_per_w
        pltpu.sync_copy(idx_hbm, idx_v)

        def g_start(step, b):
            pltpu.async_copy(
                table_hbm.at[idx_v.at[wid, pl.ds(step, 1)], pl.ds(0, DM)],
                bufs[b],
                gsems[b],
            )

        def g_wait(step, b):
            pltpu.make_async_copy(
                table_hbm.at[idx_v.at[wid, pl.ds(step, 1)], pl.ds(0, DM)],
                bufs[b],
                gsems[b],
            ).wait()

        def out_at(step):
            return out_hbm.at[pl.ds(base + step, 1), pl.ds(0, DM)]

        def w_start(step, b):
            pltpu.async_copy(bufs[b], out_at(step), wsems[b])

        def w_wait(step, b):
            pltpu.make_async_copy(bufs[b], out_at(step), wsems[b]).wait()

        for j in range(NBUF):
            g_start(j, j)

        LAG = NBUF // 2

        def body(i4, carry):
            for b in range(NBUF):
                step = i4 * NBUF + b
                g_wait(step, b)
                w_start(step, b)

                @pl.when(step >= LAG)
                def _():
                    j = step - LAG
                    jb = (b - LAG) % NBUF
                    w_wait(j, jb)

                    @pl.when(step + NBUF - LAG < b_per_w)
                    def _():
                        g_start(j + NBUF, jb)

            return carry

        lax.fori_loop(0, b_per_w // NBUF, body, 0)

        for j in range(b_per_w - LAG, b_per_w):
            w_wait(j, j % NBUF)

    return gather_kernel(idx2d, cell_means)


def _tc_gather_tail(idx, cell_means, out_main, B, V, D, DM):
    DT = LANES

    def tail_kernel(idx_ref, tab_ref, _, o_ref):
        ids = idx_ref[:, 0]
        onehot = (
            ids[:, None] == lax.broadcasted_iota(jnp.int32, (B, V), 1)
        ).astype(jnp.float32)
        o_ref[...] = jnp.dot(
            onehot,
            tab_ref[...],
            preferred_element_type=jnp.float32,
            precision=lax.Precision.HIGHEST,
        )

    return pl.pallas_call(
        tail_kernel,
        grid=(1,),
        in_specs=[
            pl.BlockSpec((B, 1), lambda g: (0, 0)),
            pl.BlockSpec((V, DT), lambda g: (0, DM // DT)),
            pl.BlockSpec(memory_space=pl.ANY),
        ],
        out_specs=pl.BlockSpec((B, DT), lambda g: (0, DM // DT)),
        out_shape=jax.ShapeDtypeStruct((B, D), jnp.float32),
        input_output_aliases={2: 0},
    )(idx.reshape(B, 1), cell_means, out_main)


def kernel(cell_index, cell_means):
    B = cell_index.shape[0]
    V, D = cell_means.shape
    DM = (D // LANES) * LANES

    idx = cell_index.astype(jnp.int32)
    idx2d = idx.reshape(NW, B // NW)

    out_main = _sc_gather_main(idx2d, cell_means, B, D, DM)
    if DM == D:
        return out_main
    return _tc_gather_tail(idx, cell_means, out_main, B, V, D, DM)
